# Initial kernel scaffold; baseline (speedup 1.0000x reference)
import functools

import jax
import jax.numpy as jnp
from jax import lax
from jax.experimental import pallas as pl
from jax.experimental.pallas import tpu as pltpu
from jax.experimental.pallas import tpu_sc as plsc

_B = 16384
_L = 200
_EMBED = 16
_TOTAL = _B * _L
_NC = 2
_NS = 16
_NW = _NC * _NS
_PER_W = _TOTAL // _NW
_CHUNK = 2048
_NCHUNKS = _PER_W // _CHUNK


def _build():
    mesh = plsc.VectorSubcoreMesh(core_axis_name="c", subcore_axis_name="s")

    @functools.partial(
        pl.kernel,
        out_type=jax.ShapeDtypeStruct((_TOTAL, _EMBED), jnp.float32),
        mesh=mesh,
        scratch_types=[
            pltpu.VMEM((_CHUNK,), jnp.int32),
            pltpu.VMEM((_CHUNK, _EMBED), jnp.float32),
            pltpu.SemaphoreType.DMA,
        ],
    )
    def emb_gather(idx_hbm, table_hbm, out_hbm, idx_v, rows_v, sem):
        wid = lax.axis_index("s") * _NC + lax.axis_index("c")
        base = wid * _PER_W

        @functools.partial(pl.loop, 0, _NCHUNKS)
        def _chunk(g):
            off = base + g * _CHUNK
            pltpu.sync_copy(idx_hbm.at[pl.ds(off, _CHUNK)], idx_v)
            pltpu.async_copy(table_hbm.at[idx_v], rows_v, sem).wait()
            pltpu.sync_copy(rows_v, out_hbm.at[pl.ds(off, _CHUNK)])

    return emb_gather


_emb_gather = _build()


@jax.jit
def kernel(log_seqs, item_emb_weight):
    idx = log_seqs.reshape(_TOTAL)
    out = _emb_gather(idx, item_emb_weight)
    return out.reshape(_B, _L, _EMBED)


# speedup vs baseline: 2.4854x; 2.4854x over previous
import functools

import jax
import jax.numpy as jnp
from jax import lax
from jax.experimental import pallas as pl
from jax.experimental.pallas import tpu as pltpu
from jax.experimental.pallas import tpu_sc as plsc

_B = 16384
_L = 200
_EMBED = 16
_TOTAL = _B * _L
_NC = 2
_NS = 16
_NW = _NC * _NS
_PER_W = _TOTAL // _NW
_CHUNK = 2048
_NCHUNKS = _PER_W // _CHUNK


def _build():
    mesh = plsc.VectorSubcoreMesh(core_axis_name="c", subcore_axis_name="s")

    @functools.partial(
        pl.kernel,
        out_type=jax.ShapeDtypeStruct((_TOTAL, _EMBED), jnp.float32),
        mesh=mesh,
        scratch_types=[
            pltpu.VMEM((_CHUNK,), jnp.int32),
            pltpu.VMEM((_CHUNK, _EMBED), jnp.float32),
            pltpu.SemaphoreType.DMA,
        ],
        compiler_params=pltpu.CompilerParams(use_tc_tiling_on_sc=False),
    )
    def emb_gather(idx_hbm, table_hbm, out_hbm, idx_v, rows_v, sem):
        wid = lax.axis_index("s") * _NC + lax.axis_index("c")
        base = wid * _PER_W

        @pl.loop(0, _NCHUNKS)
        def _chunk(g):
            off = base + g * _CHUNK
            pltpu.sync_copy(idx_hbm.at[pl.ds(off, _CHUNK)], idx_v)
            pltpu.async_copy(table_hbm.at[idx_v], rows_v, sem).wait()
            pltpu.sync_copy(rows_v, out_hbm.at[pl.ds(off, _CHUNK)])

    return emb_gather


_emb_gather = _build()


@jax.jit
def kernel(log_seqs, item_emb_weight):
    idx = log_seqs.reshape(_TOTAL)
    out = _emb_gather(idx, item_emb_weight)
    return out.reshape(_B, _L, _EMBED)


# speedup vs baseline: 2.5605x; 1.0302x over previous
import functools

import jax
import jax.numpy as jnp
from jax import lax
from jax.experimental import pallas as pl
from jax.experimental.pallas import tpu as pltpu
from jax.experimental.pallas import tpu_sc as plsc

_B = 16384
_L = 200
_EMBED = 16
_TOTAL = _B * _L
_NC = 2
_NS = 16
_NW = _NC * _NS
_PER_W = _TOTAL // _NW
_CHUNK = 2048
_NCHUNKS = _PER_W // _CHUNK


def _build():
    mesh = plsc.VectorSubcoreMesh(core_axis_name="c", subcore_axis_name="s")

    @functools.partial(
        pl.kernel,
        out_type=jax.ShapeDtypeStruct((_TOTAL, _EMBED), jnp.float32),
        mesh=mesh,
        scratch_types=[
            pltpu.VMEM((2, _CHUNK), jnp.int32),
            pltpu.VMEM((2, _CHUNK, _EMBED), jnp.float32),
            pltpu.SemaphoreType.DMA,
        ],
        compiler_params=pltpu.CompilerParams(use_tc_tiling_on_sc=False),
    )
    def emb_gather(idx_hbm, table_hbm, out_hbm, idx_v, rows_v, gsem):
        wid = lax.axis_index("s") * _NC + lax.axis_index("c")
        base = wid * _PER_W

        def fire(g, b):
            off = base + g * _CHUNK
            pltpu.sync_copy(idx_hbm.at[pl.ds(off, _CHUNK)], idx_v.at[b])
            pltpu.async_copy(table_hbm.at[idx_v.at[b]], rows_v.at[b], gsem)

        def drain(g, b):
            off = base + g * _CHUNK
            pltpu.make_async_copy(
                table_hbm.at[idx_v.at[b]], rows_v.at[b], gsem
            ).wait()
            pltpu.sync_copy(rows_v.at[b], out_hbm.at[pl.ds(off, _CHUNK)])

        fire(0, 0)

        @pl.loop(0, _NCHUNKS, step=2)
        def _chunk(g):
            fire(g + 1, 1)
            drain(g, 0)

            @pl.when(g + 2 < _NCHUNKS)
            def _():
                fire(g + 2, 0)

            drain(g + 1, 1)

    return emb_gather


_emb_gather = _build()


@jax.jit
def kernel(log_seqs, item_emb_weight):
    idx = log_seqs.reshape(_TOTAL)
    out = _emb_gather(idx, item_emb_weight)
    return out.reshape(_B, _L, _EMBED)


# speedup vs baseline: 4.9028x; 1.9148x over previous
import jax
import jax.numpy as jnp
from jax import lax
from jax.experimental import pallas as pl
from jax.experimental.pallas import tpu as pltpu
from jax.experimental.pallas import tpu_sc as plsc

_B = 16384
_L = 200
_EMBED = 16
_TOTAL = _B * _L
_NC = 2
_NS = 16
_NW = _NC * _NS
_UNIT = 2048
_NUNITS = _TOTAL // _UNIT
_PER_W = _NUNITS // _NW
_TCP = 64
_LSLAB = _NC * 128 * 8 * 128
_E8SLAB = 128 * 8 * 128


def _build():
    mesh = plsc.VectorSubcoreMesh(core_axis_name="c", subcore_axis_name="s")

    @pl.kernel(
        out_type=jax.ShapeDtypeStruct((_TOTAL * _EMBED,), jnp.float32),
        mesh=mesh,
        scratch_types=[
            pltpu.VMEM((2, _UNIT), jnp.int32),
            pltpu.VMEM((2, _UNIT, _EMBED), jnp.float32),
            pltpu.VMEM((8, 2, _UNIT), jnp.float32),
            pltpu.SemaphoreType.DMA,
            pltpu.SemaphoreType.DMA,
        ],
        compiler_params=pltpu.CompilerParams(
            use_tc_tiling_on_sc=False, needs_layout_passes=False
        ),
    )
    def emb_gather(idx_hbm, table_hbm, out_hbm, idx_v, rows_v, trans_v, gsem, osem):
        wid = lax.axis_index("s") * _NC + lax.axis_index("c")
        g0 = wid * _PER_W
        iota16 = lax.iota(jnp.int32, 16)

        def fire(g, b):
            tr = g // _TCP
            tc0 = (g % _TCP) * 2
            off = tr * (128 * 8 * 128) + tc0 * 1024
            pltpu.sync_copy(idx_hbm.at[pl.ds(off, _UNIT)], idx_v.at[b])
            pltpu.async_copy(table_hbm.at[idx_v.at[b]], rows_v.at[b], gsem)

        def wait_writes():
            for _ in range(16):
                pltpu.make_async_copy(
                    trans_v.at[0, 0], out_hbm.at[pl.ds(0, _UNIT)], osem
                ).wait()

        def process(g, b):
            pltpu.make_async_copy(
                table_hbm.at[idx_v.at[b]], rows_v.at[b], gsem
            ).wait()
            tr = g // _TCP
            tc0 = (g % _TCP) * 2
            l0 = tr * 8

            @pl.loop(0, 8)
            def _s(s):
                for e8 in range(2):

                    @pl.loop(0, 8)
                    def _e(se):
                        e = jnp.full((16,), e8 * 8 + se, jnp.int32)
                        for tcp in range(2):
                            for lg in range(8):
                                roff = tcp * 1024 + s * 128 + lg * 16
                                vec = plsc.load_gather(
                                    rows_v.at[b], [roff + iota16, e]
                                )
                                woff = tcp * 1024 + se * 128 + lg * 16
                                trans_v[s, e8, pl.ds(woff, 16)] = vec

                    q = (l0 + s) * _LSLAB + e8 * _E8SLAB + tc0 * 1024
                    pltpu.async_copy(
                        trans_v.at[s, e8], out_hbm.at[pl.ds(q, _UNIT)], osem
                    )

        fire(g0, 0)

        @pl.loop(0, _PER_W, step=2)
        def _unit(k):
            fire(g0 + k + 1, 1)

            @pl.when(k > 0)
            def _():
                wait_writes()

            process(g0 + k, 0)

            @pl.when(k + 2 < _PER_W)
            def _():
                fire(g0 + k + 2, 0)

            wait_writes()
            process(g0 + k + 1, 1)

        wait_writes()

    return emb_gather


_emb_gather = _build()


@jax.jit
def kernel(log_seqs, item_emb_weight):
    idx4 = log_seqs.reshape(128, 128, 25, 8)
    idxp = jnp.transpose(idx4, (2, 0, 3, 1)).reshape(_TOTAL)
    out = _emb_gather(idxp, item_emb_weight)
    out5 = out.reshape(200, 2, 128, 8, 128)
    return jnp.transpose(out5, (2, 4, 0, 1, 3)).reshape(_B, _L, _EMBED)


# speedup vs baseline: 6.8299x; 1.3931x over previous
import jax
import jax.numpy as jnp
from jax import lax
from jax.experimental import pallas as pl
from jax.experimental.pallas import tpu as pltpu
from jax.experimental.pallas import tpu_sc as plsc

_B = 16384
_L = 200
_EMBED = 16
_TOTAL = _B * _L
_NC = 2
_NS = 16
_NW = _NC * _NS
_UNIT = 2048
_NUNITS = _TOTAL // _UNIT
_PER_W = _NUNITS // _NW
_TCP = 64
_LSLAB = _NC * 128 * 8 * 128
_E8SLAB = 128 * 8 * 128


def _build():
    mesh = plsc.VectorSubcoreMesh(core_axis_name="c", subcore_axis_name="s")

    @pl.kernel(
        out_type=jax.ShapeDtypeStruct((_TOTAL * _EMBED,), jnp.float32),
        mesh=mesh,
        scratch_types=[
            pltpu.VMEM((2, _UNIT), jnp.int32),
            pltpu.VMEM((2, _UNIT, _EMBED), jnp.float32),
            pltpu.VMEM((8 * 2 * _UNIT,), jnp.float32),
            pltpu.SemaphoreType.DMA,
            pltpu.SemaphoreType.DMA,
        ],
        compiler_params=pltpu.CompilerParams(
            use_tc_tiling_on_sc=False, needs_layout_passes=False
        ),
    )
    def emb_gather(idx_hbm, table_hbm, out_hbm, idx_v, rows_v, trans_v, gsem, osem):
        wid = lax.axis_index("s") * _NC + lax.axis_index("c")
        g0 = wid * _PER_W
        iota16 = lax.iota(jnp.int32, 16)
        cols = [(iota16 + d) & 15 for d in range(16)]
        eoffs = [((c >> 3) << 11) + ((c & 7) << 7) for c in cols]

        def fire(g, b):
            tr = g // _TCP
            tc0 = (g % _TCP) * 2
            off = tr * (128 * 8 * 128) + tc0 * 1024
            pltpu.sync_copy(idx_hbm.at[pl.ds(off, _UNIT)], idx_v.at[b])
            pltpu.async_copy(table_hbm.at[idx_v.at[b]], rows_v.at[b], gsem)

        def wait_writes():
            for _ in range(16):
                pltpu.make_async_copy(
                    trans_v.at[pl.ds(0, _UNIT)], out_hbm.at[pl.ds(0, _UNIT)], osem
                ).wait()

        def process(g, b):
            pltpu.make_async_copy(
                table_hbm.at[idx_v.at[b]], rows_v.at[b], gsem
            ).wait()
            tr = g // _TCP
            tc0 = (g % _TCP) * 2
            l0 = tr * 8

            @pl.loop(0, 8)
            def _s(s):
                for tcp in range(2):

                    @pl.loop(0, 8)
                    def _lb(lb):
                        rbase = tcp * 1024 + s * 128 + lb * 16 + iota16
                        wbase = s * 4096 + tcp * 1024 + lb * 16 + iota16
                        for d in range(16):
                            vec = plsc.load_gather(rows_v.at[b], [rbase, cols[d]])
                            plsc.store_scatter(trans_v, [wbase + eoffs[d]], vec)

                for e8 in range(2):
                    q = (l0 + s) * _LSLAB + e8 * _E8SLAB + tc0 * 1024
                    pltpu.async_copy(
                        trans_v.at[pl.ds(s * 4096 + e8 * 2048, _UNIT)],
                        out_hbm.at[pl.ds(q, _UNIT)],
                        osem,
                    )

        fire(g0, 0)

        @pl.loop(0, _PER_W, step=2)
        def _unit(k):
            fire(g0 + k + 1, 1)

            @pl.when(k > 0)
            def _():
                wait_writes()

            process(g0 + k, 0)

            @pl.when(k + 2 < _PER_W)
            def _():
                fire(g0 + k + 2, 0)

            wait_writes()
            process(g0 + k + 1, 1)

        wait_writes()

    return emb_gather


_emb_gather = _build()


@jax.jit
def kernel(log_seqs, item_emb_weight):
    idx4 = log_seqs.reshape(128, 128, 25, 8)
    idxp = jnp.transpose(idx4, (2, 0, 3, 1)).reshape(_TOTAL)
    out = _emb_gather(idxp, item_emb_weight)
    out5 = out.reshape(200, 2, 128, 8, 128)
    return jnp.transpose(out5, (2, 4, 0, 1, 3)).reshape(_B, _L, _EMBED)
